# baseline (speedup 1.0000x reference)
import functools

import jax
import jax.numpy as jnp
from jax import lax
from jax.experimental import pallas as pl
from jax.experimental.pallas import tpu as pltpu
from jax.experimental.pallas import tpu_sc as plsc

N = 10000
E = 320000
D = 128
ALPHA = 0.5

NC = 2
NS = 16
NW = NC * NS
LANES = 16
VPD = D // LANES

EPW = E // NW
K = 40
K2 = 2 * K
NCHUNK = EPW // K
TOTCHUNK = E // K
RPS = 624
ZR = 48
REM = N - NS * RPS


def _pre_body(z_ref, x_ref, nf_ref, w_ref, h_ref):
    s = z_ref[...] + x_ref[...]
    h = lax.dot_general(s, w_ref[...], (((1,), (1,)), ((), ())),
                        preferred_element_type=jnp.float32)
    h_ref[...] = nf_ref[...] * h


def _post_body(z_ref, nf_ref, a_ref, w_ref, o_ref):
    s = nf_ref[...] * (a_ref[0] + a_ref[1])
    m = lax.dot_general(s, w_ref[...], (((1,), (0,)), ((), ())),
                        preferred_element_type=jnp.float32)
    o_ref[...] = (1.0 - ALPHA) * z_ref[...] - ALPHA * m


def _sc_body(h_hbm, ecomb_hbm, out_hbm,
             combA, combB, idxrSA, idxcSA, idxrSB, idxcSB,
             bufrA, bufcA, bufrB, bufcB,
             msgA, nmsgA, msgB, nmsgB, zbuf, acc,
             semGA, semGB, semIA, semIB, semSA, semSB):
    cid = lax.axis_index("c")
    sid = lax.axis_index("s")
    wid = sid * NC + cid

    @pl.loop(0, ZR)
    def _zero_fill(i):
        for j in range(VPD):
            zbuf[i, pl.ds(j * LANES, LANES)] = jnp.zeros((LANES,), jnp.float32)

    for b in range(RPS // ZR):
        pltpu.sync_copy(zbuf, acc.at[pl.ds(sid * RPS + b * ZR, ZR)])

    @pl.when(sid == NS - 1)
    def _zero_tail():
        pltpu.sync_copy(zbuf.at[pl.ds(0, REM)], acc.at[pl.ds(NS * RPS, REM)])

    plsc.subcore_barrier()

    gbase = wid * NCHUNK

    def load_idx(c, comb, sem):
        off = pl.multiple_of((gbase + c) * K2, 8)
        pltpu.async_copy(ecomb_hbm.at[pl.ds(off, K2)], comb, sem)

    def wait_idx(comb, sem):
        pltpu.make_async_copy(ecomb_hbm.at[pl.ds(0, K2)], comb, sem).wait()

    def start_gather(comb, bufr, bufc, sem):
        pltpu.async_copy(h_hbm.at[comb.at[pl.ds(0, K)]], bufr, sem)
        pltpu.async_copy(h_hbm.at[comb.at[pl.ds(K, K)]], bufc, sem)

    def wait_gather(comb, bufr, bufc, sem):
        pltpu.make_async_copy(h_hbm.at[comb.at[pl.ds(0, K)]], bufr, sem).wait()
        pltpu.make_async_copy(h_hbm.at[comb.at[pl.ds(K, K)]], bufc, sem).wait()

    def compute(bufr, bufc, msg, nmsg):
        @plsc.parallel_loop(0, K, unroll=8)
        def _compute(i):
            for j in range(VPD):
                sl = pl.ds(j * LANES, LANES)
                v = bufr[i, sl] - bufc[i, sl]
                m = jnp.maximum(v, 0.0)
                msg[i, sl] = m
                nmsg[i, sl] = -m

    def start_scatter(idxrS, idxcS, msg, nmsg, sem):
        pltpu.async_copy(msg, acc.at[idxrS], sem, add=True)
        pltpu.async_copy(nmsg, acc.at[idxcS], sem, add=True)

    def wait_scatter(idxrS, idxcS, msg, nmsg, sem):
        pltpu.make_async_copy(msg, acc.at[idxrS], sem).wait()
        pltpu.make_async_copy(nmsg, acc.at[idxcS], sem).wait()

    def copy_idx(comb, idxrS, idxcS):
        for o in (0, 16, K - 16):
            idxrS[pl.ds(o, LANES)] = comb[pl.ds(o, LANES)]
            idxcS[pl.ds(o, LANES)] = comb[pl.ds(K + o, LANES)]

    def phase(c, drain, pre, comb, idxrS, idxcS, bufr, bufc, msg, nmsg,
              semG, semI, semS):
        wait_gather(comb, bufr, bufc, semG)
        if drain is None:
            wait_scatter(idxrS, idxcS, msg, nmsg, semS)
        else:
            @pl.when(drain)
            def _drain():
                wait_scatter(idxrS, idxcS, msg, nmsg, semS)

        copy_idx(comb, idxrS, idxcS)
        if pre:
            load_idx(c + 2, comb, semI)
        compute(bufr, bufc, msg, nmsg)
        start_scatter(idxrS, idxcS, msg, nmsg, semS)
        if pre:
            wait_idx(comb, semI)
            start_gather(comb, bufr, bufc, semG)

    pltpu.sync_copy(ecomb_hbm.at[pl.ds(pl.multiple_of(gbase * K2, 8), K2)],
                    combA)
    pltpu.sync_copy(ecomb_hbm.at[pl.ds(pl.multiple_of((gbase + 1) * K2, 8),
                                       K2)], combB)
    start_gather(combA, bufrA, bufcA, semGA)
    start_gather(combB, bufrB, bufcB, semGB)

    @pl.loop(0, NCHUNK - 2, step=2)
    def _pair(c):
        phase(c, c > 1, True, combA, idxrSA, idxcSA, bufrA, bufcA,
              msgA, nmsgA, semGA, semIA, semSA)
        phase(c + 1, c > 1, True, combB, idxrSB, idxcSB, bufrB, bufcB,
              msgB, nmsgB, semGB, semIB, semSB)

    phase(NCHUNK - 2, None, False, combA, idxrSA, idxcSA, bufrA, bufcA,
          msgA, nmsgA, semGA, semIA, semSA)
    phase(NCHUNK - 1, None, False, combB, idxrSB, idxcSB, bufrB, bufcB,
          msgB, nmsgB, semGB, semIB, semSB)

    wait_scatter(idxrSA, idxcSA, msgA, nmsgA, semSA)
    wait_scatter(idxrSB, idxcSB, msgB, nmsgB, semSB)

    plsc.subcore_barrier()
    pltpu.sync_copy(acc.at[pl.ds(sid * RPS, RPS)],
                    out_hbm.at[cid, pl.ds(sid * RPS, RPS)])

    @pl.when(sid == NS - 1)
    def _copy_tail():
        pltpu.sync_copy(acc.at[pl.ds(NS * RPS, REM)],
                        out_hbm.at[cid, pl.ds(NS * RPS, REM)])


@functools.partial(
    pl.kernel,
    out_type=jax.ShapeDtypeStruct((NC, N, D), jnp.float32),
    mesh=plsc.VectorSubcoreMesh(core_axis_name="c", subcore_axis_name="s"),
    scratch_types=(
        [pltpu.VMEM((K2,), jnp.int32)] * 2
        + [pltpu.VMEM((K,), jnp.int32)] * 4
        + [pltpu.VMEM((K, D), jnp.float32)] * 8
        + [pltpu.VMEM((ZR, D), jnp.float32),
           pltpu.VMEM_SHARED((N, D), jnp.float32)]
        + [pltpu.SemaphoreType.DMA] * 6
    ),
)
def _sc_edge_kernel(h_hbm, ecomb_hbm, out_hbm, *rest):
    _sc_body(h_hbm, ecomb_hbm, out_hbm, *rest)


def kernel(z, x, edge_index, norm_factor, batch, W):
    del batch

    ei = edge_index.reshape(2, TOTCHUNK, K)
    ecomb = jnp.concatenate([ei[0], ei[1]], axis=1).reshape(-1)

    BN = 2000
    h = pl.pallas_call(
        _pre_body,
        grid=(N // BN,),
        in_specs=[
            pl.BlockSpec((BN, D), lambda i: (i, 0)),
            pl.BlockSpec((BN, D), lambda i: (i, 0)),
            pl.BlockSpec((BN, 1), lambda i: (i, 0)),
            pl.BlockSpec((D, D), lambda i: (0, 0)),
        ],
        out_specs=pl.BlockSpec((BN, D), lambda i: (i, 0)),
        out_shape=jax.ShapeDtypeStruct((N, D), jnp.float32),
    )(z, x, norm_factor, W)

    a = _sc_edge_kernel(h, ecomb)

    out = pl.pallas_call(
        _post_body,
        grid=(N // BN,),
        in_specs=[
            pl.BlockSpec((BN, D), lambda i: (i, 0)),
            pl.BlockSpec((BN, 1), lambda i: (i, 0)),
            pl.BlockSpec((NC, BN, D), lambda i: (0, i, 0)),
            pl.BlockSpec((D, D), lambda i: (0, 0)),
        ],
        out_specs=pl.BlockSpec((BN, D), lambda i: (i, 0)),
        out_shape=jax.ShapeDtypeStruct((N, D), jnp.float32),
    )(z, norm_factor, a, W)

    return out


# speedup vs baseline: 1.1599x; 1.1599x over previous
import functools

import jax
import jax.numpy as jnp
import numpy as np
from jax import lax
from jax.experimental import pallas as pl
from jax.experimental.pallas import tpu as pltpu
from jax.experimental.pallas import tpu_sc as plsc

N = 10000
E = 320000
D = 128
ALPHA = 0.5

NC = 2
NS = 16
NW = NC * NS
LANES = 16
VPD = D // LANES

EPW = E // NW
K = 40
NCHUNK = EPW // K
RPS = 624
ZR = 48
REM = N - NS * RPS

DW = D // 2

_PERM_SRC = np.empty((D,), np.int64)
for _t in range(D):
    _g, _u = _t // 32, _t % 32
    _PERM_SRC[_t] = _g * 32 + (_u // 2 if _u % 2 == 0 else 16 + _u // 2)
_PERM_NP = np.zeros((D, D), np.float32)
_PERM_NP[_PERM_SRC, np.arange(D)] = 1.0


def _pre_body(z_ref, x_ref, nf_ref, w_ref, p_ref, h_ref):
    s = z_ref[...] + x_ref[...]
    h = lax.dot_general(s, w_ref[...], (((1,), (1,)), ((), ())),
                        preferred_element_type=jnp.float32)
    hp = lax.dot_general(nf_ref[...] * h, p_ref[...], (((1,), (0,)), ((), ())),
                         preferred_element_type=jnp.float32)
    h_ref[...] = hp.astype(jnp.bfloat16)


def _post_body(z_ref, nf_ref, a_ref, w_ref, o_ref):
    s = nf_ref[...] * (a_ref[0] + a_ref[1])
    m = lax.dot_general(s, w_ref[...], (((1,), (0,)), ((), ())),
                        preferred_element_type=jnp.float32)
    o_ref[...] = (1.0 - ALPHA) * z_ref[...] - ALPHA * m


def _sc_body(h_hbm, row_hbm, col_hbm, out_hbm,
             idxrA, idxcA, idxrB, idxcB,
             idxrSA, idxcSA, idxrSB, idxcSB,
             bufrA, bufcA, bufrB, bufcB,
             msgA, nmsgA, msgB, nmsgB, zbuf, acc,
             semGA, semGB, semIA, semIB, semSA, semSB):
    cid = lax.axis_index("c")
    sid = lax.axis_index("s")
    wid = sid * NC + cid

    @pl.loop(0, ZR)
    def _zero_fill(i):
        for j in range(VPD):
            zbuf[i, pl.ds(j * LANES, LANES)] = jnp.zeros((LANES,), jnp.float32)

    for b in range(RPS // ZR):
        pltpu.sync_copy(zbuf, acc.at[pl.ds(sid * RPS + b * ZR, ZR)])

    @pl.when(sid == NS - 1)
    def _zero_tail():
        pltpu.sync_copy(zbuf.at[pl.ds(0, REM)], acc.at[pl.ds(NS * RPS, REM)])

    plsc.subcore_barrier()

    base = wid * EPW

    def load_idx(c, idxr, idxc, sem):
        off = pl.multiple_of(base + c * K, 8)
        pltpu.async_copy(row_hbm.at[pl.ds(off, K)], idxr, sem)
        pltpu.async_copy(col_hbm.at[pl.ds(off, K)], idxc, sem)

    def wait_idx(idxr, idxc, sem):
        pltpu.make_async_copy(row_hbm.at[pl.ds(0, K)], idxr, sem).wait()
        pltpu.make_async_copy(col_hbm.at[pl.ds(0, K)], idxc, sem).wait()

    def start_gather(idxr, idxc, bufr, bufc, sem):
        pltpu.async_copy(h_hbm.at[idxr], bufr, sem)
        pltpu.async_copy(h_hbm.at[idxc], bufc, sem)

    def wait_gather(idxr, idxc, bufr, bufc, sem):
        pltpu.make_async_copy(h_hbm.at[idxr], bufr, sem).wait()
        pltpu.make_async_copy(h_hbm.at[idxc], bufc, sem).wait()

    shift16 = jnp.full((LANES,), 16, jnp.int32)
    mask_hi = jnp.full((LANES,), -65536, jnp.int32)

    def unpack2(w):
        lo = lax.bitcast_convert_type(lax.shift_left(w, shift16), jnp.float32)
        hi = lax.bitcast_convert_type(lax.bitwise_and(w, mask_hi), jnp.float32)
        return lo, hi

    def compute(bufr, bufc, msg, nmsg):
        @plsc.parallel_loop(0, K, unroll=8)
        def _compute(i):
            for j in range(DW // LANES):
                sl = pl.ds(j * LANES, LANES)
                rlo, rhi = unpack2(bufr[i, sl])
                clo, chi = unpack2(bufc[i, sl])
                mlo = jnp.maximum(rlo - clo, 0.0)
                mhi = jnp.maximum(rhi - chi, 0.0)
                slo = pl.ds(j * 2 * LANES, LANES)
                shi = pl.ds(j * 2 * LANES + LANES, LANES)
                msg[i, slo] = mlo
                msg[i, shi] = mhi
                nmsg[i, slo] = -mlo
                nmsg[i, shi] = -mhi

    def start_scatter(idxr, idxc, msg, nmsg, sem):
        pltpu.async_copy(msg, acc.at[idxr], sem, add=True)
        pltpu.async_copy(nmsg, acc.at[idxc], sem, add=True)

    def wait_scatter(idxr, idxc, msg, nmsg, sem):
        pltpu.make_async_copy(msg, acc.at[idxr], sem).wait()
        pltpu.make_async_copy(nmsg, acc.at[idxc], sem).wait()

    def copy_idx(src, dst):
        for o in (0, 16, K - 16):
            dst[pl.ds(o, LANES)] = src[pl.ds(o, LANES)]

    pltpu.sync_copy(row_hbm.at[pl.ds(pl.multiple_of(base, 8), K)], idxrA)
    pltpu.sync_copy(col_hbm.at[pl.ds(pl.multiple_of(base, 8), K)], idxcA)
    pltpu.sync_copy(row_hbm.at[pl.ds(pl.multiple_of(base + K, 8), K)], idxrB)
    pltpu.sync_copy(col_hbm.at[pl.ds(pl.multiple_of(base + K, 8), K)], idxcB)
    start_gather(idxrA, idxcA, bufrA, bufcA, semGA)
    start_gather(idxrB, idxcB, bufrB, bufcB, semGB)

    @pl.loop(0, NCHUNK, step=2)
    def _pair(c):
        more = c + 2 < NCHUNK

        wait_gather(idxrA, idxcA, bufrA, bufcA, semGA)

        @pl.when(c > 0)
        def _drain_sa():
            wait_scatter(idxrSA, idxcSA, msgA, nmsgA, semSA)

        copy_idx(idxrA, idxrSA)
        copy_idx(idxcA, idxcSA)

        @pl.when(more)
        def _prefetch_ia():
            load_idx(c + 2, idxrA, idxcA, semIA)

        compute(bufrA, bufcA, msgA, nmsgA)
        start_scatter(idxrSA, idxcSA, msgA, nmsgA, semSA)

        @pl.when(more)
        def _launch_ga():
            wait_idx(idxrA, idxcA, semIA)
            start_gather(idxrA, idxcA, bufrA, bufcA, semGA)

        wait_gather(idxrB, idxcB, bufrB, bufcB, semGB)

        @pl.when(c > 0)
        def _drain_sb():
            wait_scatter(idxrSB, idxcSB, msgB, nmsgB, semSB)

        copy_idx(idxrB, idxrSB)
        copy_idx(idxcB, idxcSB)

        @pl.when(more)
        def _prefetch_ib():
            load_idx(c + 3, idxrB, idxcB, semIB)

        compute(bufrB, bufcB, msgB, nmsgB)
        start_scatter(idxrSB, idxcSB, msgB, nmsgB, semSB)

        @pl.when(more)
        def _launch_gb():
            wait_idx(idxrB, idxcB, semIB)
            start_gather(idxrB, idxcB, bufrB, bufcB, semGB)

    wait_scatter(idxrSA, idxcSA, msgA, nmsgA, semSA)
    wait_scatter(idxrSB, idxcSB, msgB, nmsgB, semSB)

    plsc.subcore_barrier()
    pltpu.sync_copy(acc.at[pl.ds(sid * RPS, RPS)],
                    out_hbm.at[cid, pl.ds(sid * RPS, RPS)])

    @pl.when(sid == NS - 1)
    def _copy_tail():
        pltpu.sync_copy(acc.at[pl.ds(NS * RPS, REM)],
                        out_hbm.at[cid, pl.ds(NS * RPS, REM)])


@functools.partial(
    pl.kernel,
    out_type=jax.ShapeDtypeStruct((NC, N, D), jnp.float32),
    mesh=plsc.VectorSubcoreMesh(core_axis_name="c", subcore_axis_name="s"),
    compiler_params=pltpu.CompilerParams(use_tc_tiling_on_sc=False),
    scratch_types=(
        [pltpu.VMEM((K,), jnp.int32)] * 8
        + [pltpu.VMEM((K, DW), jnp.int32)] * 4
        + [pltpu.VMEM((K, D), jnp.float32)] * 4
        + [pltpu.VMEM((ZR, D), jnp.float32),
           pltpu.VMEM_SHARED((N, D), jnp.float32)]
        + [pltpu.SemaphoreType.DMA] * 6
    ),
)
def _sc_edge_kernel(h_hbm, row_hbm, col_hbm, out_hbm, *rest):
    _sc_body(h_hbm, row_hbm, col_hbm, out_hbm, *rest)


def kernel(z, x, edge_index, norm_factor, batch, W):
    del batch
    row = edge_index[0]
    col = edge_index[1]

    BN = 2000
    h = pl.pallas_call(
        _pre_body,
        grid=(N // BN,),
        in_specs=[
            pl.BlockSpec((BN, D), lambda i: (i, 0)),
            pl.BlockSpec((BN, D), lambda i: (i, 0)),
            pl.BlockSpec((BN, 1), lambda i: (i, 0)),
            pl.BlockSpec((D, D), lambda i: (0, 0)),
            pl.BlockSpec((D, D), lambda i: (0, 0)),
        ],
        out_specs=pl.BlockSpec((BN, D), lambda i: (i, 0)),
        out_shape=jax.ShapeDtypeStruct((N, D), jnp.bfloat16),
    )(z, x, norm_factor, W, jnp.asarray(_PERM_NP))

    hbits = lax.bitcast_convert_type(h.reshape(N, DW, 2), jnp.int32)

    a = _sc_edge_kernel(hbits, row, col)

    out = pl.pallas_call(
        _post_body,
        grid=(N // BN,),
        in_specs=[
            pl.BlockSpec((BN, D), lambda i: (i, 0)),
            pl.BlockSpec((BN, 1), lambda i: (i, 0)),
            pl.BlockSpec((NC, BN, D), lambda i: (0, i, 0)),
            pl.BlockSpec((D, D), lambda i: (0, 0)),
        ],
        out_specs=pl.BlockSpec((BN, D), lambda i: (i, 0)),
        out_shape=jax.ShapeDtypeStruct((N, D), jnp.float32),
    )(z, norm_factor, a, W)

    return out
